# Initial kernel scaffold; baseline (speedup 1.0000x reference)
import jax
import jax.numpy as jnp
from jax.experimental import pallas as pl


def kernel(x, W_g, ln_g, ln_b, W1, b1, W2, b2):
    raise NotImplementedError("write your pallas kernel here")


# speedup vs baseline: 1.2530x; 1.2530x over previous
import functools
import math

import jax
import jax.numpy as jnp
from jax import lax
from jax.experimental import pallas as pl
from jax.experimental.pallas import tpu as pltpu
from jax.experimental.pallas import tpu_sc as plsc

TOPK = 2
CF = 1.25
THRESH = 0.2

NC = 2
NS = 16
NW = NC * NS
L = 16


def _router_body(n, e, cap, x_ref, wg_ref,
                 dst1_ref, dst2_ref, sk1_ref, sk2_ref, g1_ref, g2_ref,
                 aux_ref):
    x = x_ref[...]
    wg = wg_ref[...]
    logits = jnp.dot(x, wg, preferred_element_type=jnp.float32)
    mx = jnp.max(logits, axis=1, keepdims=True)
    ex = jnp.exp(logits - mx)
    se = jnp.sum(ex, axis=1, keepdims=True)
    probs = ex / se
    z = mx + jnp.log(se)

    lane = lax.broadcasted_iota(jnp.int32, (n, e), 1)
    m1 = jnp.max(probs, axis=1, keepdims=True)
    e1 = jnp.min(jnp.where(probs == m1, lane, e), axis=1, keepdims=True)
    probs_m = jnp.where(lane == e1, -1.0, probs)
    m2 = jnp.max(probs_m, axis=1, keepdims=True)
    e2 = jnp.min(jnp.where(probs_m == m2, lane, e), axis=1, keepdims=True)

    mask1 = (lane == e1).astype(jnp.float32)
    mask2 = (lane == e2).astype(jnp.float32)

    ch = 512
    nch = n // ch
    r_i = lax.broadcasted_iota(jnp.int32, (ch, ch), 0)
    c_i = lax.broadcasted_iota(jnp.int32, (ch, ch), 1)
    tri = (c_i <= r_i).astype(jnp.float32)
    carry1 = jnp.zeros((1, e), jnp.float32)
    carry2 = jnp.zeros((1, e), jnp.float32)
    c1_parts = []
    c2_parts = []
    for i in range(nch):
        m1c = lax.slice_in_dim(mask1, i * ch, (i + 1) * ch, axis=0)
        m2c = lax.slice_in_dim(mask2, i * ch, (i + 1) * ch, axis=0)
        c1_parts.append(jnp.dot(tri, m1c, preferred_element_type=jnp.float32)
                        + carry1)
        c2_parts.append(jnp.dot(tri, m2c, preferred_element_type=jnp.float32)
                        + carry2)
        carry1 = carry1 + jnp.sum(m1c, axis=0, keepdims=True)
        carry2 = carry2 + jnp.sum(m2c, axis=0, keepdims=True)
    c1 = jnp.concatenate(c1_parts, axis=0)
    c2 = jnp.concatenate(c2_parts, axis=0)

    pos1 = jnp.sum(c1 * mask1, axis=1, keepdims=True) - 1.0
    pos2 = (jnp.sum(c2 * mask2, axis=1, keepdims=True) - 1.0
            + jnp.sum(carry1 * mask2, axis=1, keepdims=True))

    keep1 = (pos1 < cap).astype(jnp.float32)
    keep2 = jnp.logical_and(pos2 < cap, m2 > THRESH).astype(jnp.float32)
    p1 = jnp.clip(pos1, 0.0, cap - 1.0).astype(jnp.int32)
    p2 = jnp.clip(pos2, 0.0, cap - 1.0).astype(jnp.int32)
    dst1 = e1 * cap + p1
    dst2 = e2 * cap + p2
    slots = e * cap
    sk1 = jnp.where(keep1 > 0.0, dst1, slots)
    sk2 = jnp.where(keep2 > 0.0, dst2, slots)

    dst1_ref[...] = dst1
    dst2_ref[...] = dst2
    sk1_ref[...] = sk1
    sk2_ref[...] = sk2
    g1_ref[...] = m1 * keep1
    g2_ref[...] = m2 * keep2

    density = jnp.sum(probs, axis=0, keepdims=True) / n
    f = carry1 / n
    bal = e * jnp.sum(f * density, keepdims=True)
    rz = jnp.sum(z * z, keepdims=True) / n
    tot = 0.01 * bal + 0.001 * rz
    rr = lax.broadcasted_iota(jnp.int32, (8, 128), 0)
    cc = lax.broadcasted_iota(jnp.int32, (8, 128), 1)
    aux_ref[...] = (bal * ((rr == 0) & (cc == 0))
                    + rz * ((rr == 0) & (cc == 1))
                    + tot * ((rr == 0) & (cc == 2)))


def _ffn_body(buf_ref, w1_ref, b1_ref, w2_ref, b2_ref, lng_ref, lnb_ref,
              y_ref):
    xb = buf_ref[...]
    mu = jnp.mean(xb, axis=1, keepdims=True)
    d = xb - mu
    var = jnp.mean(d * d, axis=1, keepdims=True)
    normed = d * lax.rsqrt(var + 1e-5) * lng_ref[0] + lnb_ref[0]
    h = jnp.dot(normed, w1_ref[0], preferred_element_type=jnp.float32)
    h = h + b1_ref[0]
    h = jnp.where(h > 0.0, h, 0.01 * h)
    y = jnp.dot(h, w2_ref[0], preferred_element_type=jnp.float32)
    y_ref[...] = y + b2_ref[0]


def kernel(x, W_g, ln_g, ln_b, W1, b1, W2, b2):
    b_, s_, d = x.shape
    n = b_ * s_
    e = W_g.shape[1]
    hid = W1.shape[2]
    cap = int(math.ceil(TOPK * n / e * CF))
    slots = e * cap

    xt = x.reshape(n, d)

    router = pl.pallas_call(
        functools.partial(_router_body, n, e, cap),
        out_shape=(
            jax.ShapeDtypeStruct((n, 1), jnp.int32),
            jax.ShapeDtypeStruct((n, 1), jnp.int32),
            jax.ShapeDtypeStruct((n, 1), jnp.int32),
            jax.ShapeDtypeStruct((n, 1), jnp.int32),
            jax.ShapeDtypeStruct((n, 1), jnp.float32),
            jax.ShapeDtypeStruct((n, 1), jnp.float32),
            jax.ShapeDtypeStruct((8, 128), jnp.float32),
        ),
    )
    dst1, dst2, sk1, sk2, g1, g2, aux = router(xt, W_g)
    dst1 = dst1.reshape(n)
    dst2 = dst2.reshape(n)
    sk1 = sk1.reshape(n)
    sk2 = sk2.reshape(n)
    g1 = g1.reshape(n)
    g2 = g2.reshape(n)

    xp = jnp.concatenate([xt, jnp.zeros((8, d), xt.dtype)], axis=0)

    mesh = plsc.VectorSubcoreMesh(core_axis_name="c", subcore_axis_name="s",
                                  num_cores=NC, num_subcores=NS)
    rows_per_w = slots // NW
    chunk = rows_per_w // 2
    tokpad = slots + L

    @functools.partial(
        pl.kernel,
        out_type=jax.ShapeDtypeStruct((slots, d), jnp.float32),
        mesh=mesh,
        scratch_types=[
            pltpu.VMEM((n,), jnp.int32),
            pltpu.VMEM((n,), jnp.int32),
            pltpu.VMEM((tokpad,), jnp.int32),
            pltpu.VMEM((chunk, d), jnp.float32),
            pltpu.SemaphoreType.DMA,
        ],
        compiler_params=pltpu.CompilerParams(needs_layout_passes=False),
    )
    def dispatch(xp_hbm, sk1_hbm, sk2_hbm, buf_hbm, sk1_v, sk2_v, tok_v,
                 rows_v, sem):
        cid = lax.axis_index("c")
        sid = lax.axis_index("s")
        wid = sid * NC + cid
        pltpu.sync_copy(sk1_hbm, sk1_v)
        pltpu.sync_copy(sk2_hbm, sk2_v)

        fill = jnp.full((L,), n, jnp.int32)

        def init_body(i, _):
            tok_v[pl.ds(i * L, L)] = fill
            return 0
        lax.fori_loop(0, tokpad // L, init_body, 0)

        def scat_body(i, _):
            base = i * L
            toks = base + lax.iota(jnp.int32, L)
            s1 = sk1_v[pl.ds(base, L)]
            plsc.store_scatter(tok_v, [s1], toks, mask=s1 < slots)
            s2 = sk2_v[pl.ds(base, L)]
            plsc.store_scatter(tok_v, [s2], toks, mask=s2 < slots)
            return 0
        lax.fori_loop(0, n // L, scat_body, 0)

        for c in range(2):
            row0 = wid * rows_per_w + c * chunk
            pltpu.async_copy(xp_hbm.at[tok_v.at[pl.ds(row0, chunk)]],
                             rows_v, sem).wait()
            pltpu.sync_copy(rows_v, buf_hbm.at[pl.ds(row0, chunk)])

    buf = dispatch(xp, sk1, sk2)

    y = pl.pallas_call(
        _ffn_body,
        grid=(e,),
        in_specs=[
            pl.BlockSpec((cap, d), lambda i: (i, 0)),
            pl.BlockSpec((1, d, hid), lambda i: (i, 0, 0)),
            pl.BlockSpec((1, 1, hid), lambda i: (i, 0, 0)),
            pl.BlockSpec((1, hid, d), lambda i: (i, 0, 0)),
            pl.BlockSpec((1, 1, d), lambda i: (i, 0, 0)),
            pl.BlockSpec((1, 1, d), lambda i: (i, 0, 0)),
            pl.BlockSpec((1, 1, d), lambda i: (i, 0, 0)),
        ],
        out_specs=pl.BlockSpec((cap, d), lambda i: (i, 0)),
        out_shape=jax.ShapeDtypeStruct((slots, d), jnp.float32),
    )(buf, W1, b1.reshape(e, 1, hid), W2, b2.reshape(e, 1, d),
      ln_g.reshape(e, 1, d), ln_b.reshape(e, 1, d))

    tok_per_w = n // NW

    @functools.partial(
        pl.kernel,
        out_type=jax.ShapeDtypeStruct((n, d), jnp.float32),
        mesh=mesh,
        scratch_types=[
            pltpu.VMEM((tok_per_w,), jnp.int32),
            pltpu.VMEM((tok_per_w,), jnp.int32),
            pltpu.VMEM((tok_per_w,), jnp.float32),
            pltpu.VMEM((tok_per_w,), jnp.float32),
            pltpu.VMEM((tok_per_w, d), jnp.float32),
            pltpu.VMEM((tok_per_w, d), jnp.float32),
            pltpu.SemaphoreType.DMA,
            pltpu.SemaphoreType.DMA,
        ],
        compiler_params=pltpu.CompilerParams(needs_layout_passes=False),
    )
    def combine(y_hbm, dst1_hbm, dst2_hbm, g1_hbm, g2_hbm, out_hbm,
                s1_v, s2_v, g1_v, g2_v, r1_v, r2_v, sem1, sem2):
        cid = lax.axis_index("c")
        sid = lax.axis_index("s")
        wid = sid * NC + cid
        t0 = wid * tok_per_w
        pltpu.sync_copy(dst1_hbm.at[pl.ds(t0, tok_per_w)], s1_v)
        pltpu.sync_copy(dst2_hbm.at[pl.ds(t0, tok_per_w)], s2_v)
        pltpu.sync_copy(g1_hbm.at[pl.ds(t0, tok_per_w)], g1_v)
        pltpu.sync_copy(g2_hbm.at[pl.ds(t0, tok_per_w)], g2_v)
        cp1 = pltpu.async_copy(y_hbm.at[s1_v], r1_v, sem1)
        cp2 = pltpu.async_copy(y_hbm.at[s2_v], r2_v, sem2)
        cp1.wait()
        cp2.wait()

        def tok_body(t, _):
            idx = jnp.full((L,), t, jnp.int32)
            gv1 = plsc.load_gather(g1_v, [idx])
            gv2 = plsc.load_gather(g2_v, [idx])
            for j in range(d // L):
                a = r1_v[t, pl.ds(j * L, L)]
                b = r2_v[t, pl.ds(j * L, L)]
                r1_v[t, pl.ds(j * L, L)] = a * gv1 + b * gv2
            return 0
        lax.fori_loop(0, tok_per_w, tok_body, 0)
        pltpu.sync_copy(r1_v, out_hbm.at[pl.ds(t0, tok_per_w)])

    out = combine(y, dst1, dst2, g1, g2)
    out = out.reshape(b_, s_, d)

    balance_loss = aux[0, 0]
    router_z_loss = aux[0, 1]
    total_aux_loss = aux[0, 2]
    return out, total_aux_loss, balance_loss, router_z_loss


# speedup vs baseline: 1.4894x; 1.1886x over previous
import functools
import math

import jax
import jax.numpy as jnp
from jax import lax
from jax.experimental import pallas as pl
from jax.experimental.pallas import tpu as pltpu
from jax.experimental.pallas import tpu_sc as plsc

TOPK = 2
CF = 1.25
THRESH = 0.2

NC = 2
NS = 16
NW = NC * NS
L = 16


def _router_body(n, e, cap, x_ref, wg_ref,
                 dst1_ref, dst2_ref, sk1_ref, sk2_ref, g1_ref, g2_ref,
                 aux_ref):
    x = x_ref[...]
    wg = wg_ref[...]
    logits = jnp.dot(x, wg, preferred_element_type=jnp.float32)
    mx = jnp.max(logits, axis=1, keepdims=True)
    ex = jnp.exp(logits - mx)
    se = jnp.sum(ex, axis=1, keepdims=True)
    probs = ex / se
    z = mx + jnp.log(se)

    lane = lax.broadcasted_iota(jnp.int32, (n, e), 1)
    m1 = jnp.max(probs, axis=1, keepdims=True)
    e1 = jnp.min(jnp.where(probs == m1, lane, e), axis=1, keepdims=True)
    probs_m = jnp.where(lane == e1, -1.0, probs)
    m2 = jnp.max(probs_m, axis=1, keepdims=True)
    e2 = jnp.min(jnp.where(probs_m == m2, lane, e), axis=1, keepdims=True)

    mask1 = (lane == e1).astype(jnp.float32)
    mask2 = (lane == e2).astype(jnp.float32)

    ch = 512
    nch = n // ch
    r_i = lax.broadcasted_iota(jnp.int32, (ch, ch), 0)
    c_i = lax.broadcasted_iota(jnp.int32, (ch, ch), 1)
    tri = (c_i <= r_i).astype(jnp.float32)
    carry1 = jnp.zeros((1, e), jnp.float32)
    carry2 = jnp.zeros((1, e), jnp.float32)
    c1_parts = []
    c2_parts = []
    for i in range(nch):
        m1c = lax.slice_in_dim(mask1, i * ch, (i + 1) * ch, axis=0)
        m2c = lax.slice_in_dim(mask2, i * ch, (i + 1) * ch, axis=0)
        c1_parts.append(jnp.dot(tri, m1c, preferred_element_type=jnp.float32)
                        + carry1)
        c2_parts.append(jnp.dot(tri, m2c, preferred_element_type=jnp.float32)
                        + carry2)
        carry1 = carry1 + jnp.sum(m1c, axis=0, keepdims=True)
        carry2 = carry2 + jnp.sum(m2c, axis=0, keepdims=True)
    c1 = jnp.concatenate(c1_parts, axis=0)
    c2 = jnp.concatenate(c2_parts, axis=0)

    pos1 = jnp.sum(c1 * mask1, axis=1, keepdims=True) - 1.0
    pos2 = (jnp.sum(c2 * mask2, axis=1, keepdims=True) - 1.0
            + jnp.sum(carry1 * mask2, axis=1, keepdims=True))

    keep1 = (pos1 < cap).astype(jnp.float32)
    keep2 = jnp.logical_and(pos2 < cap, m2 > THRESH).astype(jnp.float32)
    p1 = jnp.clip(pos1, 0.0, cap - 1.0).astype(jnp.int32)
    p2 = jnp.clip(pos2, 0.0, cap - 1.0).astype(jnp.int32)
    dst1 = e1 * cap + p1
    dst2 = e2 * cap + p2
    slots = e * cap
    sk1 = jnp.where(keep1 > 0.0, dst1, slots)
    sk2 = jnp.where(keep2 > 0.0, dst2, slots)

    dst1_ref[...] = dst1
    dst2_ref[...] = dst2
    sk1_ref[...] = sk1
    sk2_ref[...] = sk2
    g1_ref[...] = m1 * keep1
    g2_ref[...] = m2 * keep2

    density = jnp.sum(probs, axis=0, keepdims=True) / n
    f = carry1 / n
    bal = e * jnp.sum(f * density, keepdims=True)
    rz = jnp.sum(z * z, keepdims=True) / n
    tot = 0.01 * bal + 0.001 * rz
    rr = lax.broadcasted_iota(jnp.int32, (8, 128), 0)
    cc = lax.broadcasted_iota(jnp.int32, (8, 128), 1)
    aux_ref[...] = (bal * ((rr == 0) & (cc == 0))
                    + rz * ((rr == 0) & (cc == 1))
                    + tot * ((rr == 0) & (cc == 2)))


def _ffn_body(buf_ref, w1_ref, b1_ref, w2_ref, b2_ref, lng_ref, lnb_ref,
              y_ref):
    xb = buf_ref[...]
    mu = jnp.mean(xb, axis=1, keepdims=True)
    d = xb - mu
    var = jnp.mean(d * d, axis=1, keepdims=True)
    normed = d * lax.rsqrt(var + 1e-5) * lng_ref[0] + lnb_ref[0]
    h = jnp.dot(normed, w1_ref[0], preferred_element_type=jnp.float32)
    h = h + b1_ref[0]
    h = jnp.where(h > 0.0, h, 0.01 * h)
    y = jnp.dot(h, w2_ref[0], preferred_element_type=jnp.float32)
    y_ref[...] = y + b2_ref[0]


def kernel(x, W_g, ln_g, ln_b, W1, b1, W2, b2):
    b_, s_, d = x.shape
    n = b_ * s_
    e = W_g.shape[1]
    hid = W1.shape[2]
    cap = int(math.ceil(TOPK * n / e * CF))
    slots = e * cap

    xt = x.reshape(n, d)

    router = pl.pallas_call(
        functools.partial(_router_body, n, e, cap),
        out_shape=(
            jax.ShapeDtypeStruct((n, 1), jnp.int32),
            jax.ShapeDtypeStruct((n, 1), jnp.int32),
            jax.ShapeDtypeStruct((n, 1), jnp.int32),
            jax.ShapeDtypeStruct((n, 1), jnp.int32),
            jax.ShapeDtypeStruct((n, 1), jnp.float32),
            jax.ShapeDtypeStruct((n, 1), jnp.float32),
            jax.ShapeDtypeStruct((8, 128), jnp.float32),
        ),
    )
    dst1, dst2, sk1, sk2, g1, g2, aux = router(xt, W_g)
    dst1 = dst1.reshape(n)
    dst2 = dst2.reshape(n)
    sk1 = sk1.reshape(n)
    sk2 = sk2.reshape(n)
    g1 = g1.reshape(n)
    g2 = g2.reshape(n)

    mesh = plsc.VectorSubcoreMesh(core_axis_name="c", subcore_axis_name="s",
                                  num_cores=NC, num_subcores=NS)
    tok_per_w = n // NW
    bufpad = slots + cap

    @functools.partial(
        pl.kernel,
        out_type=jax.ShapeDtypeStruct((bufpad, d), jnp.float32),
        mesh=mesh,
        scratch_types=[
            pltpu.VMEM((tok_per_w,), jnp.int32),
            pltpu.VMEM((tok_per_w,), jnp.int32),
            pltpu.VMEM((tok_per_w, d), jnp.float32),
            pltpu.SemaphoreType.DMA,
            pltpu.SemaphoreType.DMA,
        ],
        compiler_params=pltpu.CompilerParams(needs_layout_passes=False),
    )
    def dispatch(x_hbm, sk1_hbm, sk2_hbm, buf_hbm, sk1_v, sk2_v, rows_v,
                 sem1, sem2):
        cid = lax.axis_index("c")
        sid = lax.axis_index("s")
        wid = sid * NC + cid
        t0 = wid * tok_per_w
        pltpu.sync_copy(sk1_hbm.at[pl.ds(t0, tok_per_w)], sk1_v)
        pltpu.sync_copy(sk2_hbm.at[pl.ds(t0, tok_per_w)], sk2_v)
        pltpu.sync_copy(x_hbm.at[pl.ds(t0, tok_per_w)], rows_v)
        c1 = pltpu.async_copy(rows_v, buf_hbm.at[sk1_v], sem1)
        c2 = pltpu.async_copy(rows_v, buf_hbm.at[sk2_v], sem2)
        c1.wait()
        c2.wait()

    buf = dispatch(xt, sk1, sk2)

    y = pl.pallas_call(
        _ffn_body,
        grid=(e,),
        in_specs=[
            pl.BlockSpec((cap, d), lambda i: (i, 0)),
            pl.BlockSpec((1, d, hid), lambda i: (i, 0, 0)),
            pl.BlockSpec((1, 1, hid), lambda i: (i, 0, 0)),
            pl.BlockSpec((1, hid, d), lambda i: (i, 0, 0)),
            pl.BlockSpec((1, 1, d), lambda i: (i, 0, 0)),
            pl.BlockSpec((1, 1, d), lambda i: (i, 0, 0)),
            pl.BlockSpec((1, 1, d), lambda i: (i, 0, 0)),
        ],
        out_specs=pl.BlockSpec((cap, d), lambda i: (i, 0)),
        out_shape=jax.ShapeDtypeStruct((slots, d), jnp.float32),
    )(buf, W1, b1.reshape(e, 1, hid), W2, b2.reshape(e, 1, d),
      ln_g.reshape(e, 1, d), ln_b.reshape(e, 1, d))

    @functools.partial(
        pl.kernel,
        out_type=jax.ShapeDtypeStruct((n, d), jnp.float32),
        mesh=mesh,
        scratch_types=[
            pltpu.VMEM((tok_per_w,), jnp.int32),
            pltpu.VMEM((tok_per_w,), jnp.int32),
            pltpu.VMEM((tok_per_w,), jnp.float32),
            pltpu.VMEM((tok_per_w,), jnp.float32),
            pltpu.VMEM((tok_per_w, d), jnp.float32),
            pltpu.VMEM((tok_per_w, d), jnp.float32),
            pltpu.SemaphoreType.DMA,
            pltpu.SemaphoreType.DMA,
        ],
        compiler_params=pltpu.CompilerParams(needs_layout_passes=False),
    )
    def combine(y_hbm, dst1_hbm, dst2_hbm, g1_hbm, g2_hbm, out_hbm,
                s1_v, s2_v, g1_v, g2_v, r1_v, r2_v, sem1, sem2):
        cid = lax.axis_index("c")
        sid = lax.axis_index("s")
        wid = sid * NC + cid
        t0 = wid * tok_per_w
        pltpu.sync_copy(dst1_hbm.at[pl.ds(t0, tok_per_w)], s1_v)
        pltpu.sync_copy(dst2_hbm.at[pl.ds(t0, tok_per_w)], s2_v)
        pltpu.sync_copy(g1_hbm.at[pl.ds(t0, tok_per_w)], g1_v)
        pltpu.sync_copy(g2_hbm.at[pl.ds(t0, tok_per_w)], g2_v)
        cp1 = pltpu.async_copy(y_hbm.at[s1_v], r1_v, sem1)
        cp2 = pltpu.async_copy(y_hbm.at[s2_v], r2_v, sem2)
        cp1.wait()
        cp2.wait()

        def tok_body(t, _):
            idx = jnp.full((L,), t, jnp.int32)
            gv1 = plsc.load_gather(g1_v, [idx])
            gv2 = plsc.load_gather(g2_v, [idx])
            p1 = gv1 != 0.0
            p2 = gv2 != 0.0
            zero = jnp.zeros((L,), jnp.float32)
            for j in range(d // L):
                a = r1_v[t, pl.ds(j * L, L)]
                b = r2_v[t, pl.ds(j * L, L)]
                va = jnp.where(p1, a * gv1, zero)
                vb = jnp.where(p2, b * gv2, zero)
                r1_v[t, pl.ds(j * L, L)] = va + vb
            return 0
        lax.fori_loop(0, tok_per_w, tok_body, 0)
        pltpu.sync_copy(r1_v, out_hbm.at[pl.ds(t0, tok_per_w)])

    out = combine(y, dst1, dst2, g1, g2)
    out = out.reshape(b_, s_, d)

    balance_loss = aux[0, 0]
    router_z_loss = aux[0, 1]
    total_aux_loss = aux[0, 2]
    return out, total_aux_loss, balance_loss, router_z_loss
